# baseline (speedup 1.0000x reference)
import functools

import numpy as np
import jax
import jax.numpy as jnp
from jax import lax
from jax.experimental import pallas as pl


def _tf2x32_np(k1, k2, x0, x1):
    k1 = np.uint32(k1)
    k2 = np.uint32(k2)
    ks = [k1, k2, np.uint32(k1 ^ k2 ^ np.uint32(0x1BD11BDA))]
    rot = (np.array([13, 15, 26, 6], np.uint32),
           np.array([17, 29, 16, 24], np.uint32))
    x = [x0.astype(np.uint32) + ks[0], x1.astype(np.uint32) + ks[1]]
    for i in range(5):
        for r in rot[i % 2]:
            a = x[0] + x[1]
            b = (x[1] << r) | (x[1] >> np.uint32(32 - r))
            x = [a, b ^ a]
        x = [x[0] + ks[(i + 1) % 3],
             x[1] + ks[(i + 2) % 3] + np.uint32(i + 1)]
    return x[0], x[1]


def _step_keys(seed, nsteps):
    key = (np.uint32(np.uint64(seed) >> np.uint64(32)),
           np.uint32(np.uint64(seed) & np.uint64(0xFFFFFFFF)))
    out = []
    for _ in range(nsteps):
        b1, b2 = _tf2x32_np(key[0], key[1],
                            np.array([0, 0], np.uint32),
                            np.array([0, 1], np.uint32))
        key = (b1[0], b2[0])
        out.append((b1[1], b2[1]))
    return out


def _rotl(x, r):
    return lax.shift_left(x, np.uint32(r)) | lax.shift_right_logical(
        x, np.uint32(32 - r))


def _threefry(k1, k2, x0, x1):
    ks0, ks1 = k1, k2
    ks2 = k1 ^ k2 ^ np.uint32(0x1BD11BDA)
    ks = (ks0, ks1, ks2)
    rot = ((13, 15, 26, 6), (17, 29, 16, 24))
    x0 = x0 + ks0
    x1 = x1 + ks1
    for i in range(5):
        for r in rot[i % 2]:
            x0 = x0 + x1
            x1 = _rotl(x1, r)
            x1 = x1 ^ x0
        x0 = x0 + ks[(i + 1) % 3]
        x1 = x1 + ks[(i + 2) % 3] + np.uint32(i + 1)
    return x0, x1


def _body(l_ref, m_ref, d_out, a_out, *, blk, V, keys):
    i = pl.program_id(0)
    l = l_ref[...]
    mraw = m_ref[...]

    m = jnp.max(l, axis=-1, keepdims=True)
    e = jnp.exp(l - m)
    s = jnp.sum(e, axis=-1, keepdims=True)
    p = e / s
    mk = (mraw > jnp.float32(0.1)).astype(jnp.float32)
    pm = p * mk
    s2 = jnp.sum(pm, axis=-1, keepdims=True)
    d = pm / s2

    row = lax.broadcasted_iota(jnp.int32, l.shape, 0)
    col = lax.broadcasted_iota(jnp.int32, l.shape, 1)
    r_glob = i * blk + row
    b_idx = lax.shift_right_logical(r_glob, 1)
    t_idx = r_glob & 1
    n = (b_idx * V + col).astype(jnp.uint32)
    (k10, k20), (k11, k21) = keys
    t0 = t_idx == 0
    k1 = jnp.where(t0, np.uint32(k10), np.uint32(k11)).astype(jnp.uint32)
    k2 = jnp.where(t0, np.uint32(k20), np.uint32(k21)).astype(jnp.uint32)
    y0, y1 = _threefry(k1, k2, jnp.zeros_like(n), n)
    bits = y0 ^ y1
    fb = lax.shift_right_logical(bits, np.uint32(9)) | np.uint32(0x3F800000)
    u = lax.bitcast_convert_type(fb, jnp.float32) - jnp.float32(1.0)
    u = jnp.maximum(jnp.float32(np.finfo(np.float32).tiny), u)
    g = -jnp.log(-jnp.log(u))

    z = jnp.log(d + jnp.float32(1e-20)) + g
    zmax = jnp.max(z, axis=-1, keepdims=True)
    idx = jnp.min(jnp.where(z == zmax, col, V), axis=-1, keepdims=True)
    dsel = jnp.sum(jnp.where(col == idx, d, jnp.float32(0.0)),
                   axis=-1, keepdims=True)
    d_out[...] = dsel
    a_out[...] = idx


def _sample_rows(lr, mr, blk, V, keys, interpret=False):
    R = lr.shape[0]
    return pl.pallas_call(
        functools.partial(_body, blk=blk, V=V, keys=keys),
        grid=(R // blk,),
        in_specs=[pl.BlockSpec((blk, V), lambda i: (i, 0)),
                  pl.BlockSpec((blk, V), lambda i: (i, 0))],
        out_specs=[pl.BlockSpec((blk, 1), lambda i: (i, 0)),
                   pl.BlockSpec((blk, 1), lambda i: (i, 0))],
        out_shape=[jax.ShapeDtypeStruct((R, 1), jnp.float32),
                   jax.ShapeDtypeStruct((R, 1), jnp.int32)],
        interpret=interpret,
    )(lr, mr)


def kernel(logits, mask_raw):
    B, T, V = logits.shape
    keys = _step_keys(42, T)
    lr = logits.reshape(B * T, V)
    mr = mask_raw.reshape(B * T, V)
    dsel, act = _sample_rows(lr, mr, 8, V, keys)
    fwd = dsel.reshape(B, T)
    action = act.reshape(B, T)[:, T - 1]
    s_dist = fwd[:, T - 1:T]
    return fwd, action, s_dist
